# baseline (speedup 1.0000x reference)
import functools

import jax
import jax.numpy as jnp
from jax import lax
from jax.experimental import pallas as pl
from jax.experimental.pallas import tpu as pltpu
from jax.experimental.pallas import tpu_sc as plsc

_INFO = plsc.get_sparse_core_info()
_NC = _INFO.num_cores
_NS = _INFO.num_subcores
_NW = _NC * _NS
_LANES = _INFO.num_lanes

_NBUF = 4
_PF = _NBUF - 1


@functools.lru_cache(maxsize=None)
def _build(B, L, D, V):
    assert B % _NW == 0 and D % _LANES == 0
    seq_per_w = B // _NW
    assert seq_per_w % _NBUF == 0
    c0 = min(128, L)
    c1 = L - c0
    nvec = D // _LANES

    mesh = plsc.VectorSubcoreMesh(core_axis_name="c", subcore_axis_name="s")

    @functools.partial(
        pl.kernel,
        out_type=jax.ShapeDtypeStruct((B, L, D), jnp.float32),
        mesh=mesh,
        compiler_params=pltpu.CompilerParams(use_tc_tiling_on_sc=False),
        scratch_types=[
            pltpu.VMEM((seq_per_w, L), jnp.int32),
            pltpu.VMEM((L, D), jnp.float32),
            [pltpu.VMEM((L, D), jnp.float32)] * _NBUF,
            [pltpu.SemaphoreType.DMA] * _NBUF,
            [pltpu.SemaphoreType.DMA] * _NBUF,
        ],
    )
    def emb_kernel(x_hbm, tok_hbm, pos_hbm, out_hbm, idx_v, pos_v,
                   tok_bufs, gsems, osems):
        cid = lax.axis_index("c")
        sid = lax.axis_index("s")
        wid = sid * _NC + cid
        seq0 = wid * seq_per_w

        pltpu.sync_copy(x_hbm.at[pl.ds(seq0, seq_per_w)], idx_v)
        pltpu.sync_copy(pos_hbm, pos_v)

        def start_gather(s, b):
            tb = tok_bufs[b]
            pltpu.async_copy(tok_hbm.at[idx_v.at[s, pl.ds(0, c0)]],
                             tb.at[pl.ds(0, c0)], gsems[b])
            if c1:
                pltpu.async_copy(tok_hbm.at[idx_v.at[s, pl.ds(c0, c1)]],
                                 tb.at[pl.ds(c0, c1)], gsems[b])

        def wait_gather(b):
            tb = tok_bufs[b]
            pltpu.make_async_copy(tok_hbm.at[idx_v.at[0, pl.ds(0, c0)]],
                                  tb.at[pl.ds(0, c0)], gsems[b]).wait()
            if c1:
                pltpu.make_async_copy(tok_hbm.at[idx_v.at[0, pl.ds(c0, c1)]],
                                      tb.at[pl.ds(c0, c1)], gsems[b]).wait()

        def start_out(s, b):
            pltpu.async_copy(tok_bufs[b], out_hbm.at[seq0 + s], osems[b])

        def wait_out(b):
            pltpu.make_async_copy(tok_bufs[b], out_hbm.at[seq0], osems[b]).wait()

        row_blk = 8
        assert L % row_blk == 0

        def add_pos(b):
            tb = tok_bufs[b]

            @pl.loop(0, L, step=row_blk)
            def _(r):
                for rr in range(row_blk):
                    for j in range(nvec):
                        sl = pl.ds(j * _LANES, _LANES)
                        plsc.addupdate(tb.at[r + rr, sl], pos_v[r + rr, sl])

        for s in range(_PF):
            start_gather(s, s % _NBUF)

        @pl.loop(0, seq_per_w, step=_NBUF)
        def _(g):
            for b in range(_NBUF):
                s = g + b
                wait_gather(b)
                add_pos(b)
                start_out(s, b)
                sp = s + _PF
                bp = (b + _PF) % _NBUF

                @pl.when(sp < seq_per_w)
                def _():
                    @pl.when(sp >= _NBUF)
                    def _():
                        wait_out(bp)

                    start_gather(sp, bp)

        for b in range(_NBUF):
            wait_out(b)

    return emb_kernel


def kernel(x, token_table, pos_table):
    B, L = x.shape
    V, D = token_table.shape
    fn = _build(B, L, D, V)
    return fn(x.astype(jnp.int32), token_table, pos_table)


# speedup vs baseline: 1.0006x; 1.0006x over previous
import functools

import jax
import jax.numpy as jnp
from jax import lax
from jax.experimental import pallas as pl
from jax.experimental.pallas import tpu as pltpu
from jax.experimental.pallas import tpu_sc as plsc

_INFO = plsc.get_sparse_core_info()
_NC = _INFO.num_cores
_NS = _INFO.num_subcores
_NW = _NC * _NS
_LANES = _INFO.num_lanes

_NBUF = 4
_PF = _NBUF - 1
_NOBUF = 2


@functools.lru_cache(maxsize=None)
def _build(B, L, D, V):
    assert B % _NW == 0 and D % _LANES == 0
    seq_per_w = B // _NW
    assert seq_per_w % _NBUF == 0
    c0 = min(128, L)
    c1 = L - c0
    nvec = D // _LANES
    row_blk = 8
    assert L % row_blk == 0

    mesh = plsc.VectorSubcoreMesh(core_axis_name="c", subcore_axis_name="s")

    @functools.partial(
        pl.kernel,
        out_type=jax.ShapeDtypeStruct((B * L * D,), jnp.float32),
        mesh=mesh,
        compiler_params=pltpu.CompilerParams(use_tc_tiling_on_sc=False),
        scratch_types=[
            pltpu.VMEM((seq_per_w * L,), jnp.int32),
            pltpu.VMEM((L, D), jnp.float32),
            [pltpu.VMEM((L, D), jnp.float32)] * _NBUF,
            [pltpu.VMEM((L * D,), jnp.float32)] * _NOBUF,
            [pltpu.SemaphoreType.DMA] * _NBUF,
            [pltpu.SemaphoreType.DMA] * _NOBUF,
        ],
    )
    def emb_kernel(x_hbm, tok_hbm, pos_hbm, out_hbm, idx_v, pos_v,
                   tok_bufs, out_bufs, gsems, osems):
        cid = lax.axis_index("c")
        sid = lax.axis_index("s")
        wid = sid * _NC + cid
        seq0 = wid * seq_per_w

        pltpu.sync_copy(x_hbm.at[pl.ds(seq0 * L, seq_per_w * L)], idx_v)
        pltpu.sync_copy(pos_hbm, pos_v)

        def start_gather(s, b):
            tb = tok_bufs[b]
            pltpu.async_copy(tok_hbm.at[idx_v.at[pl.ds(s * L, c0)]],
                             tb.at[pl.ds(0, c0)], gsems[b])
            if c1:
                pltpu.async_copy(tok_hbm.at[idx_v.at[pl.ds(s * L + c0, c1)]],
                                 tb.at[pl.ds(c0, c1)], gsems[b])

        def wait_gather(b):
            tb = tok_bufs[b]
            pltpu.make_async_copy(tok_hbm.at[idx_v.at[pl.ds(0, c0)]],
                                  tb.at[pl.ds(0, c0)], gsems[b]).wait()
            if c1:
                pltpu.make_async_copy(tok_hbm.at[idx_v.at[pl.ds(c0, c1)]],
                                      tb.at[pl.ds(c0, c1)], gsems[b]).wait()

        def start_out(s, o):
            pltpu.async_copy(out_bufs[o],
                             out_hbm.at[pl.ds((seq0 + s) * L * D, L * D)],
                             osems[o])

        def wait_out(o):
            pltpu.make_async_copy(out_bufs[o],
                                  out_hbm.at[pl.ds(0, L * D)],
                                  osems[o]).wait()

        def add_pos(b, o):
            tb = tok_bufs[b]
            ob = out_bufs[o]

            @pl.loop(0, L, step=row_blk)
            def _(r):
                for rr in range(row_blk):
                    for j in range(nvec):
                        sl = pl.ds(j * _LANES, _LANES)
                        ob[pl.ds((r + rr) * D + j * _LANES, _LANES)] = (
                            tb[r + rr, sl] + pos_v[r + rr, sl])

        for s in range(_PF):
            start_gather(s, s % _NBUF)

        @pl.loop(0, seq_per_w, step=_NBUF)
        def _(g):
            for b in range(_NBUF):
                s = g + b
                o = b % _NOBUF
                wait_gather(b)

                @pl.when(s >= _NOBUF)
                def _():
                    wait_out(o)

                add_pos(b, o)
                start_out(s, o)
                sp = s + _PF
                bp = (b + _PF) % _NBUF

                @pl.when(sp < seq_per_w)
                def _():
                    start_gather(sp, bp)

        for o in range(_NOBUF):
            wait_out(o)

    return emb_kernel


def kernel(x, token_table, pos_table):
    B, L = x.shape
    V, D = token_table.shape
    fn = _build(B, L, D, V)
    out = fn(x.astype(jnp.int32).reshape(B * L), token_table, pos_table)
    return out.reshape(B, L, D)


# speedup vs baseline: 1.1096x; 1.1089x over previous
import functools

import jax
import jax.numpy as jnp
from jax import lax
from jax.experimental import pallas as pl
from jax.experimental.pallas import tpu as pltpu
from jax.experimental.pallas import tpu_sc as plsc

_INFO = plsc.get_sparse_core_info()
_NC = _INFO.num_cores
_NS = _INFO.num_subcores
_NW = _NC * _NS
_LANES = _INFO.num_lanes

_NBUF = 2
_PADW = 128
_H0 = 104


@functools.lru_cache(maxsize=None)
def _build(B, L, D, V):
    assert B % _NW == 0 and D % _LANES == 0
    seq_per_w = B // _NW
    assert seq_per_w % _NBUF == 0
    c0 = min(128, L)
    c1 = L - c0
    nvec = D // _LANES
    row_blk = 8
    h0 = min(_H0, L)
    h1 = L - h0
    assert h0 % row_blk == 0 and h1 % row_blk == 0
    halves = ((0, h0), (h0, h1)) if h1 else ((0, h0),)

    mesh = plsc.VectorSubcoreMesh(core_axis_name="c", subcore_axis_name="s")

    @functools.partial(
        pl.kernel,
        out_type=jax.ShapeDtypeStruct((B, L, D), jnp.float32),
        mesh=mesh,
        scratch_types=[
            [pltpu.VMEM((L,), jnp.int32)] * _NBUF,
            pltpu.VMEM((L, D), jnp.float32),
            [pltpu.VMEM((L, _PADW), jnp.float32)] * _NBUF,
            [pltpu.VMEM((h, D), jnp.float32) for _, h in halves],
            [pltpu.SemaphoreType.DMA] * _NBUF,
            [pltpu.SemaphoreType.DMA] * _NBUF,
            [pltpu.SemaphoreType.DMA] * len(halves),
        ],
    )
    def emb_kernel(x_hbm, tok_hbm, pos_hbm, out_hbm, idx_bufs, pos_v,
                   tok_bufs, out_bufs, isems, gsems, osems):
        cid = lax.axis_index("c")
        sid = lax.axis_index("s")
        wid = sid * _NC + cid
        seq0 = wid * seq_per_w

        pltpu.sync_copy(pos_hbm, pos_v)

        def start_idx(s, b):
            pltpu.async_copy(x_hbm.at[pl.ds((seq0 + s) * L, L)],
                             idx_bufs[b], isems[b])

        def wait_idx(b):
            pltpu.make_async_copy(x_hbm.at[pl.ds(0, L)], idx_bufs[b],
                                  isems[b]).wait()

        def start_gather(b):
            tb = tok_bufs[b]
            iv = idx_bufs[b]
            pltpu.async_copy(tok_hbm.at[iv.at[pl.ds(0, c0)]],
                             tb.at[pl.ds(0, c0)], gsems[b])
            if c1:
                pltpu.async_copy(tok_hbm.at[iv.at[pl.ds(c0, c1)]],
                                 tb.at[pl.ds(c0, c1)], gsems[b])

        def wait_gather(b):
            tb = tok_bufs[b]
            iv = idx_bufs[b]
            pltpu.make_async_copy(tok_hbm.at[iv.at[pl.ds(0, c0)]],
                                  tb.at[pl.ds(0, c0)], gsems[b]).wait()
            if c1:
                pltpu.make_async_copy(tok_hbm.at[iv.at[pl.ds(c0, c1)]],
                                      tb.at[pl.ds(c0, c1)], gsems[b]).wait()

        def start_out(s, h):
            base, rows = halves[h]
            pltpu.async_copy(out_bufs[h],
                             out_hbm.at[seq0 + s, pl.ds(base, rows)],
                             osems[h])

        def wait_out(h):
            base, rows = halves[h]
            pltpu.make_async_copy(out_bufs[h],
                                  out_hbm.at[seq0, pl.ds(base, rows)],
                                  osems[h]).wait()

        def add_pos(b, h):
            tb = tok_bufs[b]
            ob = out_bufs[h]
            base, rows = halves[h]

            @pl.loop(0, rows, step=row_blk)
            def _(r):
                for rr in range(row_blk):
                    for j in range(nvec):
                        sl = pl.ds(j * _LANES, _LANES)
                        ob[r + rr, sl] = (tb[base + r + rr, sl]
                                          + pos_v[base + r + rr, sl])

        start_idx(0, 0)
        wait_idx(0)
        start_gather(0)
        start_idx(1, 1)

        @pl.loop(0, seq_per_w, step=_NBUF)
        def _(g):
            for b in range(_NBUF):
                s = g + b
                nb = (b + 1) % _NBUF

                @pl.when(s + 1 < seq_per_w)
                def _():
                    wait_idx(nb)
                    start_gather(nb)

                wait_gather(b)

                @pl.when(s + 2 < seq_per_w)
                def _():
                    start_idx(s + 2, b)

                for h in range(len(halves)):
                    @pl.when(s >= 1)
                    def _():
                        wait_out(h)

                    add_pos(b, h)
                    start_out(s, h)

        for h in range(len(halves)):
            wait_out(h)

    return emb_kernel


def kernel(x, token_table, pos_table):
    B, L = x.shape
    V, D = token_table.shape
    fn = _build(B, L, D, V)
    tt = jnp.pad(token_table, ((0, 0), (0, _PADW - D)))
    return fn(x.astype(jnp.int32).reshape(B * L), tt, pos_table)
